# baseline (speedup 1.0000x reference)
import jax
import jax.numpy as jnp
from jax.experimental import pallas as pl

TEMP = 50.0


def _vq_kernel(x_ref, c_ref, o_ref):
    c = c_ref[:]
    cnorm = jnp.sum(c * c, axis=1, keepdims=True)
    caug = jnp.concatenate(
        [c, jnp.ones((c.shape[0], 1), jnp.float32)], axis=1
    )
    inv_ln2 = 1.4426950408889634
    c2 = (2.0 * TEMP * inv_ln2) * c
    bias2 = (-TEMP * inv_ln2) * cnorm
    ch = c2.astype(jnp.bfloat16)
    cl = (c2 - ch.astype(jnp.float32)).astype(jnp.bfloat16)
    bh = bias2.astype(jnp.bfloat16)
    bl = (bias2 - bh.astype(jnp.float32)).astype(jnp.bfloat16)
    amat = jnp.concatenate([ch, ch, cl, bh, bl], axis=1)
    ones2 = jnp.ones((2, 512), jnp.bfloat16)
    xt = x_ref[:].reshape(512, 128).T
    bparts = []
    for g in range(32):
        x4 = xt[4 * g : 4 * g + 4, :]
        xh = x4.astype(jnp.bfloat16)
        xl = (x4 - xh.astype(jnp.float32)).astype(jnp.bfloat16)
        bparts.append(jnp.concatenate([xh, xl, xh, ones2], axis=0))
    bmat = jnp.concatenate(bparts, axis=1)
    logits = jax.lax.dot_general(
        amat,
        bmat,
        (((1,), (0,)), ((), ())),
        preferred_element_type=jnp.float32,
    )
    m = jnp.max(logits, axis=0, keepdims=True)
    e = jnp.exp2(logits - m)
    w = jax.lax.dot_general(
        caug,
        e,
        (((0,), (0,)), ((), ())),
        preferred_element_type=jnp.float32,
    )
    ratio = w[0:4, :] / w[4:5, :]
    out = jnp.concatenate(
        [ratio[:, 512 * g : 512 * (g + 1)] for g in range(32)], axis=0
    )
    o_ref[:] = out.T.reshape(8, 8192)


def kernel(x, center):
    B, F = x.shape
    out = pl.pallas_call(
        _vq_kernel,
        grid=(1,),
        in_specs=[
            pl.BlockSpec((B, F), lambda i: (0, 0)),
            pl.BlockSpec((512, 4), lambda i: (0, 0)),
        ],
        out_specs=pl.BlockSpec((B, F), lambda i: (0, 0)),
        out_shape=jax.ShapeDtypeStruct((B, F), jnp.float32),
    )(x, center)
    return out
